# initial kernel scaffold (unmeasured)
import jax
import jax.numpy as jnp
from jax import lax
from jax.experimental import pallas as pl
from jax.experimental.pallas import tpu as pltpu


def kernel(
    x,
):
    def body(*refs):
        pass

    out_shape = jax.ShapeDtypeStruct(..., jnp.float32)
    return pl.pallas_call(body, out_shape=out_shape)(...)


# baseline (device time: 104590 ns/iter reference)
import jax
import jax.numpy as jnp
from jax import lax
from jax.experimental import pallas as pl
from jax.experimental.pallas import tpu as pltpu

N_STAGES = 5


def kernel(x):
    _, _, _, m, n = x.shape

    def body(x_ref, out_ref, comm_ref, send_sems, recv_sems):
        mx = lax.axis_index("x")
        my = lax.axis_index("y")
        mz = lax.axis_index("z")

        out_ref[...] = x_ref[0, 0, 0]

        partners = [
            (mx, my, mz ^ 1),
            (mx, my, mz ^ 2),
            (mx, my ^ 1, mz),
            (mx, my ^ 2, mz),
            (mx ^ 1, my, mz),
        ]

        for s, partner in enumerate(partners):
            rdma = pltpu.make_async_remote_copy(
                src_ref=out_ref,
                dst_ref=comm_ref.at[s],
                send_sem=send_sems.at[s],
                recv_sem=recv_sems.at[s],
                device_id=partner,
                device_id_type=pl.DeviceIdType.MESH,
            )
            rdma.start()
            rdma.wait()
            out_ref[...] += comm_ref[s]

    return pl.pallas_call(
        body,
        out_shape=jax.ShapeDtypeStruct((m, n), jnp.float32),
        in_specs=[pl.BlockSpec(memory_space=pltpu.VMEM)],
        out_specs=pl.BlockSpec(memory_space=pltpu.VMEM),
        scratch_shapes=[
            pltpu.VMEM((N_STAGES, m, n), jnp.float32),
            pltpu.SemaphoreType.DMA((N_STAGES,)),
            pltpu.SemaphoreType.DMA((N_STAGES,)),
        ],
    )(x)


# device time: 46308 ns/iter; 2.2586x vs baseline; 2.2586x over previous
import jax
import jax.numpy as jnp
from jax import lax
from jax.experimental import pallas as pl
from jax.experimental.pallas import tpu as pltpu

N_STAGES = 5


def kernel(x):
    _, _, _, m, n = x.shape

    def body(x_ref, out_ref, comm_ref, send_sems, recv_sems):
        mx = lax.axis_index("x")
        my = lax.axis_index("y")
        mz = lax.axis_index("z")

        partners = [
            (mx, my, mz ^ 1),
            (mx, my ^ 1, mz),
            (mx ^ 1, my, mz),
            (mx, my, mz ^ 2),
            (mx, my ^ 2, mz),
        ]
        bits = [mz & 1, my & 1, mx, (mz >> 1) & 1, (my >> 1) & 1]

        barrier_sem = pltpu.get_barrier_semaphore()
        for p in partners:
            pl.semaphore_signal(
                barrier_sem, inc=1, device_id=p,
                device_id_type=pl.DeviceIdType.MESH,
            )
        pl.semaphore_wait(barrier_sem, N_STAGES)

        out_ref[...] = x_ref[0, 0, 0]

        start = mz * 0
        size = m
        for s in range(N_STAGES):
            half = size // 2
            b = bits[s]
            keep_start = start + b * half
            send_start = start + (1 - b) * half
            rdma = pltpu.make_async_remote_copy(
                src_ref=out_ref.at[pl.ds(send_start, half), :],
                dst_ref=comm_ref.at[s, pl.ds(0, half), :],
                send_sem=send_sems.at[s],
                recv_sem=recv_sems.at[s],
                device_id=partners[s],
                device_id_type=pl.DeviceIdType.MESH,
            )
            rdma.start()
            rdma.wait()
            out_ref[pl.ds(keep_start, half), :] += comm_ref[s, :half, :]
            start = keep_start
            size = half

        for s in reversed(range(N_STAGES)):
            b = bits[s]
            parent_start = start - b * size
            rdma = pltpu.make_async_remote_copy(
                src_ref=out_ref.at[pl.ds(start, size), :],
                dst_ref=out_ref.at[pl.ds(start, size), :],
                send_sem=send_sems.at[N_STAGES + s],
                recv_sem=recv_sems.at[N_STAGES + s],
                device_id=partners[s],
                device_id_type=pl.DeviceIdType.MESH,
            )
            rdma.start()
            rdma.wait()
            start = parent_start
            size = size * 2

    return pl.pallas_call(
        body,
        out_shape=jax.ShapeDtypeStruct((m, n), jnp.float32),
        in_specs=[pl.BlockSpec(memory_space=pltpu.VMEM)],
        out_specs=pl.BlockSpec(memory_space=pltpu.VMEM),
        scratch_shapes=[
            pltpu.VMEM((N_STAGES, m // 2, n), jnp.float32),
            pltpu.SemaphoreType.DMA((2 * N_STAGES,)),
            pltpu.SemaphoreType.DMA((2 * N_STAGES,)),
        ],
        compiler_params=pltpu.CompilerParams(collective_id=0),
    )(x)


# device time: 38659 ns/iter; 2.7055x vs baseline; 1.1979x over previous
import jax
import jax.numpy as jnp
from jax import lax
from jax.experimental import pallas as pl
from jax.experimental.pallas import tpu as pltpu

N_STAGES = 5
N_PARTS = 2


def kernel(x):
    _, _, _, m, n = x.shape
    c = n // N_PARTS

    def body(x_ref, out_ref, comm_ref, send_sems, recv_sems):
        mx = lax.axis_index("x")
        my = lax.axis_index("y")
        mz = lax.axis_index("z")

        z0 = ((mx, my, mz ^ 1), mz & 1)
        y0 = ((mx, my ^ 1, mz), my & 1)
        xx = ((mx ^ 1, my, mz), mx)
        z1 = ((mx, my, mz ^ 2), (mz >> 1) & 1)
        y1 = ((mx, my ^ 2, mz), (my >> 1) & 1)
        orders = [
            [z0, y0, xx, z1, y1],
            [y0, xx, z0, y1, z1],
        ]

        barrier_sem = pltpu.get_barrier_semaphore()
        for p, _ in orders[0]:
            pl.semaphore_signal(
                barrier_sem, inc=1, device_id=p,
                device_id_type=pl.DeviceIdType.MESH,
            )
        pl.semaphore_wait(barrier_sem, N_STAGES)

        out_ref[...] = x_ref[0, 0, 0]

        def sem_slot(phase, part, stage):
            return (phase * N_PARTS + part) * N_STAGES + stage

        starts = [mz * 0, mz * 0]
        size = m
        for k in range(N_STAGES):
            half = size // 2
            inflight = []
            for p in range(N_PARTS):
                partner, b = orders[p][k]
                keep_start = starts[p] + b * half
                send_start = starts[p] + (1 - b) * half
                rdma = pltpu.make_async_remote_copy(
                    src_ref=out_ref.at[pl.ds(send_start, half), pl.ds(p * c, c)],
                    dst_ref=comm_ref.at[k, pl.ds(0, half), pl.ds(p * c, c)],
                    send_sem=send_sems.at[sem_slot(0, p, k)],
                    recv_sem=recv_sems.at[sem_slot(0, p, k)],
                    device_id=partner,
                    device_id_type=pl.DeviceIdType.MESH,
                )
                rdma.start()
                inflight.append((rdma, p, keep_start))
            for rdma, p, keep_start in inflight:
                rdma.wait()
                out_ref[pl.ds(keep_start, half), pl.ds(p * c, c)] += (
                    comm_ref[k, pl.ds(0, half), pl.ds(p * c, c)]
                )
                starts[p] = keep_start
            size = half

        for k in reversed(range(N_STAGES)):
            inflight = []
            for p in range(N_PARTS):
                partner, b = orders[p][k]
                rdma = pltpu.make_async_remote_copy(
                    src_ref=out_ref.at[pl.ds(starts[p], size), pl.ds(p * c, c)],
                    dst_ref=out_ref.at[pl.ds(starts[p], size), pl.ds(p * c, c)],
                    send_sem=send_sems.at[sem_slot(1, p, k)],
                    recv_sem=recv_sems.at[sem_slot(1, p, k)],
                    device_id=partner,
                    device_id_type=pl.DeviceIdType.MESH,
                )
                rdma.start()
                inflight.append((rdma, p, b))
            for rdma, p, b in inflight:
                rdma.wait()
                starts[p] = starts[p] - b * size
            size = size * 2

    n_sems = 2 * N_PARTS * N_STAGES
    return pl.pallas_call(
        body,
        out_shape=jax.ShapeDtypeStruct((m, n), jnp.float32),
        in_specs=[pl.BlockSpec(memory_space=pltpu.VMEM)],
        out_specs=pl.BlockSpec(memory_space=pltpu.VMEM),
        scratch_shapes=[
            pltpu.VMEM((N_STAGES, m // 2, n), jnp.float32),
            pltpu.SemaphoreType.DMA((n_sems,)),
            pltpu.SemaphoreType.DMA((n_sems,)),
        ],
        compiler_params=pltpu.CompilerParams(collective_id=0),
    )(x)


# device time: 38575 ns/iter; 2.7113x vs baseline; 1.0022x over previous
import jax
import jax.numpy as jnp
from jax import lax
from jax.experimental import pallas as pl
from jax.experimental.pallas import tpu as pltpu

N_STAGES = 5
N_PARTS = 2


def kernel(x):
    _, _, _, m, n = x.shape
    c = n // N_PARTS

    def body(x_ref, out_ref, comm_ref, send_sems, recv_sems):
        mx = lax.axis_index("x")
        my = lax.axis_index("y")
        mz = lax.axis_index("z")

        z0 = ((mx, my, mz ^ 1), mz & 1)
        y0 = ((mx, my ^ 1, mz), my & 1)
        xx = ((mx ^ 1, my, mz), mx)
        z1 = ((mx, my, mz ^ 2), (mz >> 1) & 1)
        y1 = ((mx, my ^ 2, mz), (my >> 1) & 1)
        orders = [
            [z0, y0, xx, z1, y1],
            [y0, xx, z0, y1, z1],
        ]

        barrier_sem = pltpu.get_barrier_semaphore()
        for p, _ in orders[0]:
            pl.semaphore_signal(
                barrier_sem, inc=1, device_id=p,
                device_id_type=pl.DeviceIdType.MESH,
            )
        pl.semaphore_wait(barrier_sem, N_STAGES)

        out_ref[...] = x_ref[0, 0, 0]

        def sem_slot(phase, part, stage):
            return (phase * N_PARTS + part) * N_STAGES + stage

        def mk_rs(p, k, start, half):
            partner, b = orders[p][k]
            keep_start = start + b * half
            send_start = start + (1 - b) * half
            rdma = pltpu.make_async_remote_copy(
                src_ref=out_ref.at[pl.ds(send_start, half), pl.ds(p * c, c)],
                dst_ref=comm_ref.at[k, pl.ds(0, half), pl.ds(p * c, c)],
                send_sem=send_sems.at[sem_slot(0, p, k)],
                recv_sem=recv_sems.at[sem_slot(0, p, k)],
                device_id=partner,
                device_id_type=pl.DeviceIdType.MESH,
            )
            return rdma, keep_start

        zero = mz * 0
        rdmas = [None] * N_PARTS
        keeps = [zero] * N_PARTS
        for p in range(N_PARTS):
            rdmas[p], keeps[p] = mk_rs(p, 0, zero, m // 2)
            rdmas[p].start()
        for k in range(N_STAGES):
            half = m >> (k + 1)
            q = half // 2
            pend = []
            for p in range(N_PARTS):
                rdmas[p].wait()
                keep = keeps[p]
                if k < N_STAGES - 1:
                    _, bn = orders[p][k + 1]
                    send_off = (1 - bn) * q
                    out_ref[pl.ds(keep + send_off, q), pl.ds(p * c, c)] += (
                        comm_ref[k, pl.ds(send_off, q), pl.ds(p * c, c)]
                    )
                    rdmas[p], keeps[p] = mk_rs(p, k + 1, keep, q)
                    rdmas[p].start()
                    pend.append((p, keep, bn * q))
                else:
                    out_ref[pl.ds(keep, half), pl.ds(p * c, c)] += (
                        comm_ref[k, pl.ds(0, half), pl.ds(p * c, c)]
                    )
            for p, keep, keep_off in pend:
                out_ref[pl.ds(keep + keep_off, q), pl.ds(p * c, c)] += (
                    comm_ref[k, pl.ds(keep_off, q), pl.ds(p * c, c)]
                )

        segs = keeps
        size = m >> N_STAGES
        for k in reversed(range(N_STAGES)):
            inflight = []
            for p in range(N_PARTS):
                partner, b = orders[p][k]
                rdma = pltpu.make_async_remote_copy(
                    src_ref=out_ref.at[pl.ds(segs[p], size), pl.ds(p * c, c)],
                    dst_ref=out_ref.at[pl.ds(segs[p], size), pl.ds(p * c, c)],
                    send_sem=send_sems.at[sem_slot(1, p, k)],
                    recv_sem=recv_sems.at[sem_slot(1, p, k)],
                    device_id=partner,
                    device_id_type=pl.DeviceIdType.MESH,
                )
                rdma.start()
                inflight.append((rdma, p, b))
            for rdma, p, b in inflight:
                rdma.wait()
                segs[p] = segs[p] - b * size
            size = size * 2

    n_sems = 2 * N_PARTS * N_STAGES
    return pl.pallas_call(
        body,
        out_shape=jax.ShapeDtypeStruct((m, n), jnp.float32),
        in_specs=[pl.BlockSpec(memory_space=pltpu.VMEM)],
        out_specs=pl.BlockSpec(memory_space=pltpu.VMEM),
        scratch_shapes=[
            pltpu.VMEM((N_STAGES, m // 2, n), jnp.float32),
            pltpu.SemaphoreType.DMA((n_sems,)),
            pltpu.SemaphoreType.DMA((n_sems,)),
        ],
        compiler_params=pltpu.CompilerParams(collective_id=0),
    )(x)
